# baseline (speedup 1.0000x reference)
import functools

import jax
import jax.numpy as jnp
from jax import lax
from jax.experimental import pallas as pl
from jax.experimental.pallas import tpu as pltpu
from jax.experimental.pallas import tpu_sc as plsc

_L = 16
_W = 128


def _make_lookup(n_rows, n_workers, chunk_rows):
    rows_per_w = n_rows // n_workers
    n_chunks = rows_per_w // chunk_rows
    mesh = plsc.VectorSubcoreMesh(core_axis_name="c", subcore_axis_name="s")

    @functools.partial(
        pl.kernel,
        mesh=mesh,
        out_type=jax.ShapeDtypeStruct((n_rows, _W), jnp.int32),
        scratch_types=[
            pltpu.VMEM((_L,), jnp.int32),
            pltpu.VMEM((n_chunks, chunk_rows), jnp.int32),
            [pltpu.VMEM((chunk_rows, _W), jnp.int32) for _ in range(2)],
            [pltpu.VMEM((chunk_rows, _W), jnp.int32) for _ in range(2)],
            [pltpu.SemaphoreType.DMA for _ in range(2)],
            [pltpu.SemaphoreType.DMA for _ in range(2)],
        ],
        compiler_params=pltpu.CompilerParams(needs_layout_passes=False),
    )
    def lookup(idx_hbm, vals_hbm, out_hbm, tab_v, rid_v, in_b, out_b,
               in_sem, out_sem):
        wid = lax.axis_index("s") * 2 + lax.axis_index("c")
        pltpu.sync_copy(vals_hbm, tab_v)
        base_row = wid * rows_per_w
        lane = lax.iota(jnp.int32, _L)

        @plsc.parallel_loop(0, n_chunks, unroll=1)
        def _(c):
            for k in range(chunk_rows // _L):
                rid_v[c, pl.ds(k * _L, _L)] = (
                    base_row + c * chunk_rows + k * _L + lane)

        def start_in(c, b):
            return pltpu.async_copy(
                idx_hbm.at[rid_v.at[c]], in_b[b], in_sem[b])

        def start_out(c, b):
            return pltpu.async_copy(
                out_b[b], out_hbm.at[rid_v.at[c]], out_sem[b])

        def compute(src, dst):
            @plsc.parallel_loop(0, chunk_rows, unroll=2)
            def _(r):
                for k in range(_W // _L):
                    s = pl.ds(k * _L, _L)
                    dst[r, s] = plsc.load_gather(tab_v, [src[r, s]])

        in_cp = [None, None]
        out_cp = [None, None]
        in_cp[0] = start_in(0, 0)
        for c in range(n_chunks):
            b = c % 2
            if c + 1 < n_chunks:
                nb = (c + 1) % 2
                in_cp[nb] = start_in(c + 1, nb)
            in_cp[b].wait()
            if c >= 2:
                out_cp[b].wait()
            compute(in_b[b], out_b[b])
            out_cp[b] = start_out(c, b)
        for c in (n_chunks - 2, n_chunks - 1):
            if c >= 0:
                out_cp[c % 2].wait()

    return lookup


def kernel(inputs, vals):
    shape = inputs.shape
    x = inputs.reshape(-1, _W).astype(jnp.int32)
    tab = jnp.pad(vals.astype(jnp.int32), (0, _L - vals.shape[0]))
    out = _make_lookup(x.shape[0], 32, 80)(x, tab)
    return out.reshape(shape)


# speedup vs baseline: 1.5831x; 1.5831x over previous
import functools

import jax
import jax.numpy as jnp
from jax import lax
from jax.experimental import pallas as pl
from jax.experimental.pallas import tpu as pltpu
from jax.experimental.pallas import tpu_sc as plsc

_L = 16
_W = 128


def _make_sc_lookup(n_rows, n_cols, n_workers, chunk_rows):
    rows_per_w = n_rows // n_workers
    n_chunks = rows_per_w // chunk_rows
    mesh = plsc.VectorSubcoreMesh(core_axis_name="c", subcore_axis_name="s")

    @functools.partial(
        pl.kernel,
        mesh=mesh,
        out_type=jax.ShapeDtypeStruct((n_rows, n_cols), jnp.int32),
        scratch_types=[
            pltpu.VMEM((_L,), jnp.int32),
            pltpu.VMEM((n_chunks, chunk_rows), jnp.int32),
            [pltpu.VMEM((chunk_rows, _W), jnp.int32) for _ in range(2)],
            [pltpu.VMEM((chunk_rows, _W), jnp.int32) for _ in range(2)],
            [pltpu.SemaphoreType.DMA for _ in range(2)],
            [pltpu.SemaphoreType.DMA for _ in range(2)],
        ],
        compiler_params=pltpu.CompilerParams(needs_layout_passes=False),
    )
    def lookup(idx_hbm, vals_hbm, out_hbm, tab_v, rid_v, in_b, out_b,
               in_sem, out_sem):
        wid = lax.axis_index("s") * 2 + lax.axis_index("c")
        pltpu.sync_copy(vals_hbm, tab_v)
        base_row = wid * rows_per_w
        lane = lax.iota(jnp.int32, _L)

        @plsc.parallel_loop(0, n_chunks, unroll=1)
        def _(c):
            for k in range(chunk_rows // _L):
                rid_v[c, pl.ds(k * _L, _L)] = (
                    base_row + c * chunk_rows + k * _L + lane)

        def start_in(c, b):
            return pltpu.async_copy(
                idx_hbm.at[rid_v.at[c], pl.ds(0, _W)], in_b[b], in_sem[b])

        def start_out(c, b):
            return pltpu.async_copy(
                out_b[b], out_hbm.at[rid_v.at[c], pl.ds(0, _W)], out_sem[b])

        def compute(src, dst):
            @plsc.parallel_loop(0, chunk_rows, unroll=2)
            def _(r):
                for k in range(_W // _L):
                    s = pl.ds(k * _L, _L)
                    dst[r, s] = plsc.load_gather(tab_v, [src[r, s]])

        in_cp = [None, None]
        out_cp = [None, None]
        in_cp[0] = start_in(0, 0)
        for c in range(n_chunks):
            b = c % 2
            if c + 1 < n_chunks:
                nb = (c + 1) % 2
                in_cp[nb] = start_in(c + 1, nb)
            in_cp[b].wait()
            if c >= 2:
                out_cp[b].wait()
            compute(in_b[b], out_b[b])
            out_cp[b] = start_out(c, b)
        for c in (n_chunks - 2, n_chunks - 1):
            if c >= 0:
                out_cp[c % 2].wait()

    return lookup


def _tc_remap_body(x_ref, vals_ref, _donor_ref, out_ref):
    xb = x_ref[...]
    acc = jnp.full(xb.shape, vals_ref[0], jnp.int32)
    for k in range(1, 8):
        acc = jnp.where(xb == k, vals_ref[k], acc)
    out_ref[...] = acc


def _make_tc_remap(n_rows, n_cols, block_rows):
    return pl.pallas_call(
        _tc_remap_body,
        grid=(n_rows // block_rows,),
        in_specs=[
            pl.BlockSpec((block_rows, _W), lambda i: (i, 1)),
            pl.BlockSpec(memory_space=pltpu.SMEM),
            pl.BlockSpec(memory_space=pl.ANY),
        ],
        out_specs=pl.BlockSpec((block_rows, _W), lambda i: (i, 1)),
        out_shape=jax.ShapeDtypeStruct((n_rows, n_cols), jnp.int32),
        input_output_aliases={2: 0},
    )


def kernel(inputs, vals):
    x = inputs.astype(jnp.int32)
    n_rows, n_cols = x.shape
    tab = jnp.pad(vals.astype(jnp.int32), (0, _L - vals.shape[0]))
    sc_out = _make_sc_lookup(n_rows, n_cols, 32, 128)(x, tab)
    return _make_tc_remap(n_rows, n_cols, 2048)(x, vals.astype(jnp.int32),
                                                sc_out)


# speedup vs baseline: 2.8112x; 1.7758x over previous
import functools

import jax
import jax.numpy as jnp
from jax import lax
from jax.experimental import pallas as pl
from jax.experimental.pallas import tpu as pltpu
from jax.experimental.pallas import tpu_sc as plsc

_L = 16
_TS = 8


def _make_lookup(n_trows, n_cols, n_workers):
    cols_per_w = n_cols // n_workers
    mesh = plsc.VectorSubcoreMesh(core_axis_name="c", subcore_axis_name="s")

    @functools.partial(
        pl.kernel,
        mesh=mesh,
        out_type=jax.ShapeDtypeStruct((n_trows, _TS, n_cols), jnp.int32),
        scratch_types=[
            pltpu.VMEM((_L,), jnp.int32),
            pltpu.VMEM((2, _L), jnp.int32),
            [pltpu.VMEM((1, _TS, cols_per_w), jnp.int32) for _ in range(2)],
            [pltpu.VMEM((1, _TS, cols_per_w), jnp.int32) for _ in range(2)],
            [pltpu.SemaphoreType.DMA for _ in range(2)],
            [pltpu.SemaphoreType.DMA for _ in range(2)],
        ],
        compiler_params=pltpu.CompilerParams(needs_layout_passes=False),
    )
    def lookup(idx_hbm, vals_hbm, out_hbm, tab_v, rid_v, in_b, out_b,
               in_sem, out_sem):
        wid = lax.axis_index("s") * 2 + lax.axis_index("c")
        pltpu.sync_copy(vals_hbm, tab_v)
        col0 = wid * cols_per_w

        lane = lax.iota(jnp.int32, _L)
        rid_v[0, :] = lane
        rid_v[1, :] = lane + _L

        def start_in(c, b):
            return pltpu.async_copy(
                idx_hbm.at[rid_v.at[c // _L, pl.ds(c % _L, 1)], :,
                           pl.ds(col0, cols_per_w)],
                in_b[b], in_sem[b])

        def start_out(c, b):
            return pltpu.async_copy(
                out_b[b],
                out_hbm.at[rid_v.at[c // _L, pl.ds(c % _L, 1)], :,
                           pl.ds(col0, cols_per_w)],
                out_sem[b])

        def compute(src, dst):
            @plsc.parallel_loop(0, _TS * (cols_per_w // _L), unroll=8)
            def _(v):
                s = v // (cols_per_w // _L)
                k = v % (cols_per_w // _L)
                sl = pl.ds(k * _L, _L)
                dst[0, s, sl] = plsc.load_gather(tab_v, [src[0, s, sl]])

        in_cp = [None, None]
        out_cp = [None, None]
        in_cp[0] = start_in(0, 0)
        for c in range(n_trows):
            b = c % 2
            if c + 1 < n_trows:
                nb = (c + 1) % 2
                in_cp[nb] = start_in(c + 1, nb)
            in_cp[b].wait()
            if c >= 2:
                out_cp[b].wait()
            compute(in_b[b], out_b[b])
            out_cp[b] = start_out(c, b)
        for c in (n_trows - 2, n_trows - 1):
            if c >= 0:
                out_cp[c % 2].wait()

    return lookup


def kernel(inputs, vals):
    n_rows, n_cols = inputs.shape
    x = inputs.astype(jnp.int32).T.reshape(n_cols // _TS, _TS, n_rows)
    tab = jnp.pad(vals.astype(jnp.int32), (0, _L - vals.shape[0]))
    out = _make_lookup(n_cols // _TS, n_rows, 32)(x, tab)
    return out.reshape(n_cols, n_rows).T


# speedup vs baseline: 3.1288x; 1.1130x over previous
import functools

import jax
import jax.numpy as jnp
from jax import lax
from jax.experimental import pallas as pl
from jax.experimental.pallas import tpu as pltpu
from jax.experimental.pallas import tpu_sc as plsc

_L = 16
_TS = 8


def _make_lookup(n_trows, n_cols, n_workers):
    cols_per_w = n_cols // n_workers
    mesh = plsc.VectorSubcoreMesh(core_axis_name="c", subcore_axis_name="s")

    @functools.partial(
        pl.kernel,
        mesh=mesh,
        out_type=jax.ShapeDtypeStruct((n_trows, _TS, n_cols), jnp.int32),
        scratch_types=[
            pltpu.VMEM((_L,), jnp.int32),
            pltpu.VMEM((2, _L), jnp.int32),
            [pltpu.VMEM((1, _TS, cols_per_w), jnp.int32) for _ in range(3)],
            [pltpu.VMEM((1, _TS, cols_per_w), jnp.int32) for _ in range(2)],
            [pltpu.SemaphoreType.DMA for _ in range(3)],
            [pltpu.SemaphoreType.DMA for _ in range(2)],
        ],
        compiler_params=pltpu.CompilerParams(needs_layout_passes=False),
    )
    def lookup(idx_hbm, vals_hbm, out_hbm, tab_v, rid_v, in_b, out_b,
               in_sem, out_sem):
        wid = lax.axis_index("s") * 2 + lax.axis_index("c")
        pltpu.sync_copy(vals_hbm, tab_v)
        col0 = wid * cols_per_w

        lane = lax.iota(jnp.int32, _L)
        for j in range(2):
            t = lane + (j * _L + wid)
            t = jnp.where(t >= n_trows, t - n_trows, t)
            t = jnp.where(t >= n_trows, t - n_trows, t)
            rid_v[j, :] = t

        def rid_at(c):
            return rid_v.at[c // _L, pl.ds(c % _L, 1)]

        def start_in(c, b):
            return pltpu.async_copy(
                idx_hbm.at[rid_at(c), :, pl.ds(col0, cols_per_w)],
                in_b[b], in_sem[b])

        def start_out(c, b):
            return pltpu.async_copy(
                out_b[b],
                out_hbm.at[rid_at(c), :, pl.ds(col0, cols_per_w)],
                out_sem[b])

        def compute(src, dst):
            @plsc.parallel_loop(0, _TS * (cols_per_w // _L), unroll=8)
            def _(v):
                s = v // (cols_per_w // _L)
                k = v % (cols_per_w // _L)
                sl = pl.ds(k * _L, _L)
                dst[0, s, sl] = plsc.load_gather(tab_v, [src[0, s, sl]])

        in_cp = [None, None, None]
        out_cp = [None, None]
        in_cp[0] = start_in(0, 0)
        in_cp[1] = start_in(1, 1)
        for c in range(n_trows):
            b = c % 3
            ob = c % 2
            if c + 2 < n_trows:
                nb = (c + 2) % 3
                in_cp[nb] = start_in(c + 2, nb)
            in_cp[b].wait()
            if c >= 2:
                out_cp[ob].wait()
            compute(in_b[b], out_b[ob])
            out_cp[ob] = start_out(c, ob)
        for c in (n_trows - 2, n_trows - 1):
            if c >= 0:
                out_cp[c % 2].wait()

    return lookup


def kernel(inputs, vals):
    n_rows, n_cols = inputs.shape
    x = inputs.astype(jnp.int32).T.reshape(n_cols // _TS, _TS, n_rows)
    tab = jnp.pad(vals.astype(jnp.int32), (0, _L - vals.shape[0]))
    out = _make_lookup(n_cols // _TS, n_rows, 32)(x, tab)
    return out.reshape(n_cols, n_rows).T
